# baseline (speedup 1.0000x reference)
import functools

import jax
import jax.numpy as jnp
from jax import lax
from jax.experimental import pallas as pl
from jax.experimental.pallas import tpu as pltpu
from jax.experimental.pallas import tpu_sc as plsc

E = 8
D_MODEL = 1024
D_EXPERT = 1024
T_TOK = 2048
A_TOT = 2 * T_TOK
BLK = 256
NT = A_TOT // BLK + E - 1
S_ROWS = NT * BLK
NC = 2
NS = 16
NW = NC * NS
TPW = T_TOK // NW
HTOK = TPW // 2
CH = 512


def _router_kernel(x_ref, gamma_ref, beta_ref, wg_ref,
                   xn_ref, probs_ref, loss_ref, s0_ref, s1_ref,
                   w0_ref, w1_ref, te_ref):
    x = x_ref[...]
    mean = jnp.mean(x, axis=-1, keepdims=True)
    var = jnp.mean((x - mean) ** 2, axis=-1, keepdims=True)
    xn = (x - mean) / jnp.sqrt(var + 1e-5) * gamma_ref[...][None, :] + beta_ref[...][None, :]
    xn_ref[...] = xn

    logits = jax.lax.dot_general(
        xn, wg_ref[...], (((1,), (1,)), ((), ())),
        preferred_element_type=jnp.float32)
    m = jnp.max(logits, axis=-1, keepdims=True)
    p = jnp.exp(logits - m)
    probs = p / jnp.sum(p, axis=-1, keepdims=True)
    probs_ref[...] = probs

    T = probs.shape[0]
    e_iota = jax.lax.broadcasted_iota(jnp.int32, (T, E), 1)
    m0 = jnp.max(probs, axis=-1, keepdims=True)
    i0 = jnp.min(jnp.where(probs == m0, e_iota, E), axis=-1, keepdims=True)
    oh0 = (e_iota == i0)
    pm = jnp.where(oh0, -1.0, probs)
    m1 = jnp.max(pm, axis=-1, keepdims=True)
    i1 = jnp.min(jnp.where(pm == m1, e_iota, E), axis=-1, keepdims=True)
    oh1 = (e_iota == i1)
    denom = m0 + m1 + 1e-8
    w0_ref[...] = jnp.broadcast_to(m0 / denom, (T, 128))
    w1_ref[...] = jnp.broadcast_to(m1 / denom, (T, 128))

    oh0f = oh0.astype(jnp.float32)
    oh1f = oh1.astype(jnp.float32)

    oh_all = jnp.concatenate([oh0f, oh1f], axis=0)
    r_i = jax.lax.broadcasted_iota(jnp.int32, (CH, CH), 0)
    c_i = jax.lax.broadcasted_iota(jnp.int32, (CH, CH), 1)
    tri = (r_i >= c_i).astype(jnp.float32)
    carry = jnp.zeros((1, E), jnp.float32)
    parts = []
    for k in range(A_TOT // CH):
        chunk = oh_all[k * CH:(k + 1) * CH]
        cum = jax.lax.dot_general(
            tri, chunk, (((1,), (0,)), ((), ())),
            preferred_element_type=jnp.float32) + carry
        carry = cum[CH - 1:CH, :]
        parts.append(cum)
    cum_all = jnp.concatenate(parts, axis=0)
    counts = carry

    padded = jnp.floor((counts + (BLK - 1)) / BLK) * BLK
    lt8 = (jax.lax.broadcasted_iota(jnp.int32, (E, E), 0)
           < jax.lax.broadcasted_iota(jnp.int32, (E, E), 1)).astype(jnp.float32)
    offs = jax.lax.dot_general(
        padded, lt8, (((1,), (0,)), ((), ())),
        preferred_element_type=jnp.float32)

    rank0 = jnp.sum(cum_all[:T] * oh0f, axis=-1, keepdims=True) - 1.0
    rank1 = jnp.sum(cum_all[T:] * oh1f, axis=-1, keepdims=True) - 1.0
    obc0 = jnp.sum(offs * oh0f, axis=-1, keepdims=True)
    obc1 = jnp.sum(offs * oh1f, axis=-1, keepdims=True)
    s0 = (obc0 + rank0).astype(jnp.int32)
    s1 = (obc1 + rank1).astype(jnp.int32)
    s0_ref[...] = jnp.broadcast_to(s0, (T, 128))
    s1_ref[...] = jnp.broadcast_to(s1, (T, 128))

    ends = (offs + padded) / BLK
    i8 = (jax.lax.broadcasted_iota(jnp.int32, (E, E), 0)
          == jax.lax.broadcasted_iota(jnp.int32, (E, E), 1)).astype(jnp.float32)
    ends_col = jax.lax.dot_general(
        i8, ends, (((1,), (1,)), ((), ())),
        preferred_element_type=jnp.float32)
    n_row = jax.lax.broadcasted_iota(jnp.int32, (1, 128), 1).astype(jnp.float32)
    te = jnp.sum((n_row >= ends_col).astype(jnp.float32), axis=0, keepdims=True)
    te_ref[...] = jnp.minimum(te, E - 1).astype(jnp.int32)

    psum = jnp.sum(probs, axis=0)
    csum = jnp.sum(oh0f + oh1f, axis=0)
    loss = 0.01 * E * jnp.sum(csum * psum) / (T * T)
    loss_ref[...] = jnp.full((1, 128), loss, jnp.float32)


def _dispatch_scatter(xn_hbm, s0_hbm, s1_hbm, xg_hbm,
                      idx0_v, idx1_v, rows_v, sem):
    wid = lax.axis_index("s") * NC + lax.axis_index("c")
    base = wid * TPW
    pltpu.sync_copy(s0_hbm.at[wid], idx0_v)
    pltpu.sync_copy(s1_hbm.at[wid], idx1_v)
    pltpu.sync_copy(xn_hbm.at[pl.ds(base, TPW)], rows_v)
    pltpu.async_copy(rows_v, xg_hbm.at[idx0_v], sem).wait()
    pltpu.async_copy(rows_v, xg_hbm.at[idx1_v], sem).wait()


def _ffn_grouped_kernel(te_ref, xg_ref, w1_ref, b1_ref, w2_ref, b2_ref, yo_ref):
    del te_ref
    xg = xg_ref[...].astype(jnp.bfloat16)
    w1 = w1_ref[0].astype(jnp.bfloat16)
    h = jax.lax.dot_general(
        xg, w1, (((1,), (1,)), ((), ())),
        preferred_element_type=jnp.float32) + b1_ref[0]
    h = 0.5 * h * (1.0 + jax.lax.erf(h * 0.7071067811865476))
    w2 = w2_ref[0].astype(jnp.bfloat16)
    yo_ref[...] = jax.lax.dot_general(
        h.astype(jnp.bfloat16), w2, (((1,), (1,)), ((), ())),
        preferred_element_type=jnp.float32) + b2_ref[0]


def _combine_gather(x_hbm, yo_hbm, s0_hbm, s1_hbm, w0_hbm, w1_hbm, out_hbm,
                    idx0_v, idx1_v, xb_v, y0_v, y1_v, wb0_v, wb1_v, sem0, sem1):
    wid = lax.axis_index("s") * NC + lax.axis_index("c")
    base = wid * TPW
    pltpu.sync_copy(s0_hbm.at[wid], idx0_v)
    pltpu.sync_copy(s1_hbm.at[wid], idx1_v)
    pltpu.sync_copy(w0_hbm.at[pl.ds(base, TPW)], wb0_v)
    pltpu.sync_copy(w1_hbm.at[pl.ds(base, TPW)], wb1_v)
    for h in range(2):
        t0 = base + h * HTOK
        pltpu.sync_copy(x_hbm.at[pl.ds(t0, HTOK)], xb_v)
        cp0 = pltpu.async_copy(yo_hbm.at[idx0_v.at[h]], y0_v, sem0)
        cp1 = pltpu.async_copy(yo_hbm.at[idx1_v.at[h]], y1_v, sem1)
        cp0.wait()
        cp1.wait()

        def body(t, acc, h=h):
            wv0 = wb0_v[h * HTOK + t, pl.ds(0, 16)]
            wv1 = wb1_v[h * HTOK + t, pl.ds(0, 16)]
            for sl in range(8):
                for c in range(8):
                    d = pl.ds(c * 16, 16)
                    xb_v[t, sl, d] = (xb_v[t, sl, d]
                                      + wv0 * y0_v[t, sl, d]
                                      + wv1 * y1_v[t, sl, d])
            return acc

        lax.fori_loop(0, HTOK, body, 0)
        pltpu.sync_copy(xb_v, out_hbm.at[pl.ds(t0, HTOK)])


def _sc_mesh():
    return plsc.VectorSubcoreMesh(core_axis_name="c", subcore_axis_name="s",
                                  num_cores=NC, num_subcores=NS)


def _make_scatter_call():
    return pl.kernel(
        _dispatch_scatter,
        out_type=jax.ShapeDtypeStruct((S_ROWS, 8, 128), jnp.float32),
        mesh=_sc_mesh(),
        scratch_types=[
            pltpu.VMEM((TPW,), jnp.int32),
            pltpu.VMEM((TPW,), jnp.int32),
            pltpu.VMEM((TPW, 8, 128), jnp.float32),
            pltpu.SemaphoreType.DMA,
        ],
    )


def _make_combine_call():
    return pl.kernel(
        _combine_gather,
        out_type=jax.ShapeDtypeStruct((T_TOK, 8, 128), jnp.float32),
        mesh=_sc_mesh(),
        scratch_types=[
            pltpu.VMEM((2, HTOK), jnp.int32),
            pltpu.VMEM((2, HTOK), jnp.int32),
            pltpu.VMEM((HTOK, 8, 128), jnp.float32),
            pltpu.VMEM((HTOK, 8, 128), jnp.float32),
            pltpu.VMEM((HTOK, 8, 128), jnp.float32),
            pltpu.VMEM((TPW, 128), jnp.float32),
            pltpu.VMEM((TPW, 128), jnp.float32),
            pltpu.SemaphoreType.DMA,
            pltpu.SemaphoreType.DMA,
        ],
    )


def kernel(x, gamma, beta, Wg, W1, b1, W2, b2):
    Bq, T, D = x.shape
    n_tok = Bq * T
    x_flat = x.reshape(n_tok, D)

    xn, probs, loss, s0x, s1x, w0x, w1x, te = pl.pallas_call(
        _router_kernel,
        out_shape=(
            jax.ShapeDtypeStruct((n_tok, D), jnp.float32),
            jax.ShapeDtypeStruct((n_tok, E), jnp.float32),
            jax.ShapeDtypeStruct((1, 128), jnp.float32),
            jax.ShapeDtypeStruct((n_tok, 128), jnp.int32),
            jax.ShapeDtypeStruct((n_tok, 128), jnp.int32),
            jax.ShapeDtypeStruct((n_tok, 128), jnp.float32),
            jax.ShapeDtypeStruct((n_tok, 128), jnp.float32),
            jax.ShapeDtypeStruct((1, 128), jnp.int32),
        ),
    )(x_flat, gamma, beta, Wg)

    s0c = s0x[:, 0]
    s1c = s1x[:, 0]

    xg = _make_scatter_call()(xn.reshape(n_tok, 8, 128),
                              s0c.reshape(NW, TPW), s1c.reshape(NW, TPW))

    yo = pl.pallas_call(
        _ffn_grouped_kernel,
        grid_spec=pltpu.PrefetchScalarGridSpec(
            num_scalar_prefetch=1,
            grid=(NT,),
            in_specs=[
                pl.BlockSpec((BLK, D), lambda n, te: (n, 0)),
                pl.BlockSpec((1, D_EXPERT, D), lambda n, te: (te[n], 0, 0)),
                pl.BlockSpec((1, 1, D_EXPERT), lambda n, te: (te[n], 0, 0)),
                pl.BlockSpec((1, D, D_EXPERT), lambda n, te: (te[n], 0, 0)),
                pl.BlockSpec((1, 1, D), lambda n, te: (te[n], 0, 0)),
            ],
            out_specs=pl.BlockSpec((BLK, D), lambda n, te: (n, 0)),
        ),
        out_shape=jax.ShapeDtypeStruct((S_ROWS, D), jnp.float32),
    )(te[0, :NT], xg.reshape(S_ROWS, D), W1, b1.reshape(E, 1, D_EXPERT),
      W2, b2.reshape(E, 1, D))

    out = _make_combine_call()(x_flat.reshape(n_tok, 8, 128),
                        yo.reshape(S_ROWS, 8, 128),
                        s0c.reshape(NW, 2, HTOK), s1c.reshape(NW, 2, HTOK),
                        w0x, w1x)

    return out.reshape(Bq, T, D), loss[0, 0], probs


# speedup vs baseline: 1.0147x; 1.0147x over previous
import functools

import jax
import jax.numpy as jnp
from jax import lax
from jax.experimental import pallas as pl
from jax.experimental.pallas import tpu as pltpu
from jax.experimental.pallas import tpu_sc as plsc

E = 8
D_MODEL = 1024
D_EXPERT = 1024
T_TOK = 2048
A_TOT = 2 * T_TOK
BLK = 256
NT = A_TOT // BLK + E - 1
S_ROWS = NT * BLK
NC = 2
NS = 16
NW = NC * NS
TPW = T_TOK // NW
HTOK = TPW // 2
CH = 512


def _router_kernel(x_ref, gamma_ref, beta_ref, wg_ref,
                   xn_ref, probs_ref, loss_ref, s0_ref, s1_ref,
                   w0_ref, w1_ref, te_ref):
    x = x_ref[...]
    mean = jnp.mean(x, axis=-1, keepdims=True)
    var = jnp.mean((x - mean) ** 2, axis=-1, keepdims=True)
    xn = (x - mean) / jnp.sqrt(var + 1e-5) * gamma_ref[...][None, :] + beta_ref[...][None, :]
    xn_ref[...] = xn

    logits = jax.lax.dot_general(
        xn, wg_ref[...], (((1,), (1,)), ((), ())),
        preferred_element_type=jnp.float32)
    m = jnp.max(logits, axis=-1, keepdims=True)
    p = jnp.exp(logits - m)
    probs = p / jnp.sum(p, axis=-1, keepdims=True)
    probs_ref[...] = probs

    T = probs.shape[0]
    e_iota = jax.lax.broadcasted_iota(jnp.int32, (T, E), 1)
    m0 = jnp.max(probs, axis=-1, keepdims=True)
    i0 = jnp.min(jnp.where(probs == m0, e_iota, E), axis=-1, keepdims=True)
    oh0 = (e_iota == i0)
    pm = jnp.where(oh0, -1.0, probs)
    m1 = jnp.max(pm, axis=-1, keepdims=True)
    i1 = jnp.min(jnp.where(pm == m1, e_iota, E), axis=-1, keepdims=True)
    oh1 = (e_iota == i1)
    denom = m0 + m1 + 1e-8
    w0_ref[...] = jnp.broadcast_to(m0 / denom, (T, 128))
    w1_ref[...] = jnp.broadcast_to(m1 / denom, (T, 128))

    oh0f = oh0.astype(jnp.float32)
    oh1f = oh1.astype(jnp.float32)

    oh_all = jnp.concatenate([oh0f, oh1f], axis=0)
    r_i = jax.lax.broadcasted_iota(jnp.int32, (CH, CH), 0)
    c_i = jax.lax.broadcasted_iota(jnp.int32, (CH, CH), 1)
    tri = (r_i >= c_i).astype(jnp.float32)
    carry = jnp.zeros((1, E), jnp.float32)
    parts = []
    for k in range(A_TOT // CH):
        chunk = oh_all[k * CH:(k + 1) * CH]
        cum = jax.lax.dot_general(
            tri, chunk, (((1,), (0,)), ((), ())),
            preferred_element_type=jnp.float32) + carry
        carry = cum[CH - 1:CH, :]
        parts.append(cum)
    cum_all = jnp.concatenate(parts, axis=0)
    counts = carry

    padded = jnp.floor((counts + (BLK - 1)) / BLK) * BLK
    lt8 = (jax.lax.broadcasted_iota(jnp.int32, (E, E), 0)
           < jax.lax.broadcasted_iota(jnp.int32, (E, E), 1)).astype(jnp.float32)
    offs = jax.lax.dot_general(
        padded, lt8, (((1,), (0,)), ((), ())),
        preferred_element_type=jnp.float32)

    rank0 = jnp.sum(cum_all[:T] * oh0f, axis=-1, keepdims=True) - 1.0
    rank1 = jnp.sum(cum_all[T:] * oh1f, axis=-1, keepdims=True) - 1.0
    obc0 = jnp.sum(offs * oh0f, axis=-1, keepdims=True)
    obc1 = jnp.sum(offs * oh1f, axis=-1, keepdims=True)

    cmat = (jax.lax.broadcasted_iota(jnp.int32, (T, 128), 0) % 128
            == jax.lax.broadcasted_iota(jnp.int32, (T, 128), 1)).astype(jnp.float32)
    rmat = (jax.lax.broadcasted_iota(jnp.int32, (T // 128, T), 1) // 128
            == jax.lax.broadcasted_iota(jnp.int32, (T // 128, T), 0)).astype(jnp.float32)

    def pack_col(v):
        return jax.lax.dot_general(
            rmat, v * cmat, (((1,), (0,)), ((), ())),
            precision=jax.lax.Precision.HIGHEST,
            preferred_element_type=jnp.float32)

    s0_ref[...] = pack_col(obc0 + rank0).astype(jnp.int32)
    s1_ref[...] = pack_col(obc1 + rank1).astype(jnp.int32)

    ends = (offs + padded) / BLK
    i8 = (jax.lax.broadcasted_iota(jnp.int32, (E, E), 0)
          == jax.lax.broadcasted_iota(jnp.int32, (E, E), 1)).astype(jnp.float32)
    ends_col = jax.lax.dot_general(
        i8, ends, (((1,), (1,)), ((), ())),
        preferred_element_type=jnp.float32)
    n_row = jax.lax.broadcasted_iota(jnp.int32, (1, 128), 1).astype(jnp.float32)
    te = jnp.sum((n_row >= ends_col).astype(jnp.float32), axis=0, keepdims=True)
    te_ref[...] = jnp.minimum(te, E - 1).astype(jnp.int32)

    psum = jnp.sum(probs, axis=0)
    csum = jnp.sum(oh0f + oh1f, axis=0)
    loss = 0.01 * E * jnp.sum(csum * psum) / (T * T)
    loss_ref[...] = jnp.full((1, 128), loss, jnp.float32)


def _dispatch_scatter(xn_hbm, s0_hbm, s1_hbm, xg_hbm,
                      idx0_v, idx1_v, rows_v, sem):
    wid = lax.axis_index("s") * NC + lax.axis_index("c")
    base = wid * TPW
    pltpu.sync_copy(s0_hbm.at[pl.ds(base, TPW)], idx0_v)
    pltpu.sync_copy(s1_hbm.at[pl.ds(base, TPW)], idx1_v)
    pltpu.sync_copy(xn_hbm.at[pl.ds(base, TPW)], rows_v)
    pltpu.async_copy(rows_v, xg_hbm.at[idx0_v], sem).wait()
    pltpu.async_copy(rows_v, xg_hbm.at[idx1_v], sem).wait()


def _ffn_grouped_kernel(te_ref, xg_ref, w1_ref, b1_ref, w2_ref, b2_ref, yo_ref):
    del te_ref
    xg = xg_ref[...].astype(jnp.bfloat16)
    w1 = w1_ref[0].astype(jnp.bfloat16)
    h = jax.lax.dot_general(
        xg, w1, (((1,), (1,)), ((), ())),
        preferred_element_type=jnp.float32) + b1_ref[0]
    h = 0.5 * h * (1.0 + jax.lax.erf(h * 0.7071067811865476))
    w2 = w2_ref[0].astype(jnp.bfloat16)
    yo_ref[...] = jax.lax.dot_general(
        h.astype(jnp.bfloat16), w2, (((1,), (1,)), ((), ())),
        preferred_element_type=jnp.float32) + b2_ref[0]


def _combine_gather(x_hbm, yo_hbm, s0_hbm, s1_hbm, w0_hbm, w1_hbm, out_hbm,
                    idx0_v, idx1_v, xb_v, y0_v, y1_v, wb0_v, wb1_v,
                    sem0, sem1):
    wid = lax.axis_index("s") * NC + lax.axis_index("c")
    base = wid * TPW
    pltpu.sync_copy(s0_hbm.at[pl.ds(base, TPW)], idx0_v)
    pltpu.sync_copy(s1_hbm.at[pl.ds(base, TPW)], idx1_v)
    pltpu.sync_copy(w0_hbm.at[pl.ds(base, TPW)], wb0_v)
    pltpu.sync_copy(w1_hbm.at[pl.ds(base, TPW)], wb1_v)
    for h in range(2):
        t0 = base + h * HTOK
        pltpu.sync_copy(x_hbm.at[pl.ds(t0, HTOK)], xb_v)
        cp0 = pltpu.async_copy(yo_hbm.at[idx0_v.at[pl.ds(h * HTOK, HTOK)]],
                               y0_v, sem0)
        cp1 = pltpu.async_copy(yo_hbm.at[idx1_v.at[pl.ds(h * HTOK, HTOK)]],
                               y1_v, sem1)
        cp0.wait()
        cp1.wait()

        def body(t, acc, h=h):
            wv0 = wb0_v[h * HTOK + t, pl.ds(0, 16)]
            wv1 = wb1_v[h * HTOK + t, pl.ds(0, 16)]
            for sl in range(8):
                for c in range(8):
                    d = pl.ds(c * 16, 16)
                    xb_v[t, sl, d] = (xb_v[t, sl, d]
                                      + wv0 * y0_v[t, sl, d]
                                      + wv1 * y1_v[t, sl, d])
            return acc

        lax.fori_loop(0, HTOK, body, 0)
        pltpu.sync_copy(xb_v, out_hbm.at[pl.ds(t0, HTOK)])


def _sc_mesh():
    return plsc.VectorSubcoreMesh(core_axis_name="c", subcore_axis_name="s",
                                  num_cores=NC, num_subcores=NS)


def _make_scatter_call():
    return pl.kernel(
        _dispatch_scatter,
        out_type=jax.ShapeDtypeStruct((S_ROWS, 8, 128), jnp.float32),
        mesh=_sc_mesh(),
        scratch_types=[
            pltpu.VMEM((TPW,), jnp.int32),
            pltpu.VMEM((TPW,), jnp.int32),
            pltpu.VMEM((TPW, 8, 128), jnp.float32),
            pltpu.SemaphoreType.DMA,
        ],
    )


def _make_combine_call():
    return pl.kernel(
        _combine_gather,
        out_type=jax.ShapeDtypeStruct((T_TOK, 8, 128), jnp.float32),
        mesh=_sc_mesh(),
        scratch_types=[
            pltpu.VMEM((TPW,), jnp.int32),
            pltpu.VMEM((TPW,), jnp.int32),
            pltpu.VMEM((HTOK, 8, 128), jnp.float32),
            pltpu.VMEM((HTOK, 8, 128), jnp.float32),
            pltpu.VMEM((HTOK, 8, 128), jnp.float32),
            pltpu.VMEM((TPW, 128), jnp.float32),
            pltpu.VMEM((TPW, 128), jnp.float32),
            pltpu.SemaphoreType.DMA,
            pltpu.SemaphoreType.DMA,
        ],
    )


def kernel(x, gamma, beta, Wg, W1, b1, W2, b2):
    Bq, T, D = x.shape
    n_tok = Bq * T
    x_flat = x.reshape(n_tok, D)

    xn, probs, loss, s0p, s1p, w0x, w1x, te = pl.pallas_call(
        _router_kernel,
        out_shape=(
            jax.ShapeDtypeStruct((n_tok, D), jnp.float32),
            jax.ShapeDtypeStruct((n_tok, E), jnp.float32),
            jax.ShapeDtypeStruct((1, 128), jnp.float32),
            jax.ShapeDtypeStruct((n_tok // 128, 128), jnp.int32),
            jax.ShapeDtypeStruct((n_tok // 128, 128), jnp.int32),
            jax.ShapeDtypeStruct((n_tok, 128), jnp.float32),
            jax.ShapeDtypeStruct((n_tok, 128), jnp.float32),
            jax.ShapeDtypeStruct((1, 128), jnp.int32),
        ),
    )(x_flat, gamma, beta, Wg)

    s0f = s0p.reshape(n_tok)
    s1f = s1p.reshape(n_tok)

    xg = _make_scatter_call()(xn.reshape(n_tok, 8, 128), s0f, s1f)

    yo = pl.pallas_call(
        _ffn_grouped_kernel,
        grid_spec=pltpu.PrefetchScalarGridSpec(
            num_scalar_prefetch=1,
            grid=(NT,),
            in_specs=[
                pl.BlockSpec((BLK, D), lambda n, te: (n, 0)),
                pl.BlockSpec((1, D_EXPERT, D), lambda n, te: (te[0, n], 0, 0)),
                pl.BlockSpec((1, 1, D_EXPERT), lambda n, te: (te[0, n], 0, 0)),
                pl.BlockSpec((1, D, D_EXPERT), lambda n, te: (te[0, n], 0, 0)),
                pl.BlockSpec((1, 1, D), lambda n, te: (te[0, n], 0, 0)),
            ],
            out_specs=pl.BlockSpec((BLK, D), lambda n, te: (n, 0)),
        ),
        out_shape=jax.ShapeDtypeStruct((S_ROWS, D), jnp.float32),
    )(te, xg.reshape(S_ROWS, D), W1, b1.reshape(E, 1, D_EXPERT),
      W2, b2.reshape(E, 1, D))

    out = _make_combine_call()(x_flat.reshape(n_tok, 8, 128),
                               yo.reshape(S_ROWS, 8, 128),
                               s0f, s1f, w0x, w1x)

    return out.reshape(Bq, T, D), loss[0, 0], probs
